# baseline (speedup 1.0000x reference)
import functools

import jax
import jax.numpy as jnp
from jax import lax
from jax.experimental import pallas as pl
from jax.experimental.pallas import tpu as pltpu
from jax.experimental.pallas import tpu_sc as plsc

_VOCAB = 100000
_EMBED = 64
_SEQ = 200

_NC = 2
_NS = 16
_NW = _NC * _NS
_LANES = 16

_IDX_MINOR = 100
_CHUNK_ROWS = 400
_CR = _CHUNK_ROWS // _IDX_MINOR


def _sc_body(n_rows, wt_hbm, wp_hbm, idx_hbm, out_hbm,
             pos_v, idx_v, rows_v, sem):
    per_w = n_rows // _NW
    n_chunks = per_w // _CHUNK_ROWS
    idx_rows_per_w = per_w // _IDX_MINOR

    wid = lax.axis_index("s") * _NC + lax.axis_index("c")

    pltpu.sync_copy(wp_hbm.at[pl.ds(0, _SEQ)], pos_v)

    def chunk_body(c, carry):
        row_base = wid * per_w + c * _CHUNK_ROWS
        irow = wid * idx_rows_per_w + c * _CR

        pltpu.sync_copy(idx_hbm.at[pl.ds(irow, _CR)], idx_v)

        for rep in range(_CHUNK_ROWS // _SEQ):
            pltpu.sync_copy(wp_hbm.at[pl.ds(0, _SEQ)],
                            rows_v.at[pl.ds(rep * _SEQ, _SEQ)])

        copies = []
        for i in range(_CR):
            copies.append(pltpu.async_copy(
                wt_hbm.at[idx_v.at[i]],
                rows_v.at[pl.ds(i * _IDX_MINOR, _IDX_MINOR)],
                sem, add=True))
        for d in copies:
            d.wait()

        pltpu.sync_copy(rows_v, out_hbm.at[pl.ds(row_base, _CHUNK_ROWS)])
        return carry

    lax.fori_loop(0, n_chunks, chunk_body, 0)


def kernel(input, W_tok, W_pos):
    batch, seq = input.shape
    n_rows = batch * seq
    ids_flat = input.reshape(n_rows // _IDX_MINOR, _IDX_MINOR).astype(jnp.int32)

    mesh = plsc.VectorSubcoreMesh(core_axis_name="c", subcore_axis_name="s",
                                  num_cores=_NC, num_subcores=_NS)
    out = pl.kernel(
        functools.partial(_sc_body, n_rows),
        out_type=jax.ShapeDtypeStruct((n_rows, _EMBED), jnp.float32),
        mesh=mesh,
        scratch_types=[
            pltpu.VMEM((_SEQ, _EMBED), jnp.float32),
            pltpu.VMEM((_CR, _IDX_MINOR), jnp.int32),
            pltpu.VMEM((_CHUNK_ROWS, _EMBED), jnp.float32),
            pltpu.SemaphoreType.DMA,
        ],
        compiler_params=pltpu.CompilerParams(use_tc_tiling_on_sc=False),
    )(W_tok, W_pos, ids_flat)
    return out.reshape(batch, seq, _EMBED)


# speedup vs baseline: 1.5649x; 1.5649x over previous
import functools

import jax
import jax.numpy as jnp
from jax import lax
from jax.experimental import pallas as pl
from jax.experimental.pallas import tpu as pltpu
from jax.experimental.pallas import tpu_sc as plsc

_VOCAB = 100000
_EMBED = 64
_SEQ = 200

_NC = 2
_NS = 16
_NW = _NC * _NS
_LANES = 16

_IDX_MINOR = 100
_CHUNK_ROWS = 800
_CR = _CHUNK_ROWS // _IDX_MINOR
_REPS = _CHUNK_ROWS // _SEQ


def _sc_body(n_rows, wt_hbm, wp_hbm, idx_hbm, out_hbm,
             pos_v, idx_a, idx_b, rows_a, rows_b,
             sem_ia, sem_ib, sem_ga, sem_gb, sem_oa, sem_ob):
    per_w = n_rows // _NW
    n_chunks = per_w // _CHUNK_ROWS
    idx_rows_per_w = per_w // _IDX_MINOR

    wid = lax.axis_index("s") * _NC + lax.axis_index("c")

    def idx_copy(c, idx_v, sem):
        irow = wid * idx_rows_per_w + c * _CR
        return pltpu.make_async_copy(idx_hbm.at[pl.ds(irow, _CR)], idx_v, sem)

    def gather_copies(idx_v, rows_v, sem):
        return [pltpu.make_async_copy(
                    wt_hbm.at[idx_v.at[i]],
                    rows_v.at[pl.ds(i * _IDX_MINOR, _IDX_MINOR)],
                    sem)
                for i in range(_CR)]

    def out_copy(c, rows_v, sem):
        row_base = wid * per_w + c * _CHUNK_ROWS
        return pltpu.make_async_copy(
            rows_v, out_hbm.at[pl.ds(row_base, _CHUNK_ROWS)], sem)

    def add_pos(rows_v):
        def jbody(j, carry):
            pos_regs = [pos_v[j, pl.ds(l * _LANES, _LANES)]
                        for l in range(_EMBED // _LANES)]
            for rep in range(_REPS):
                r = rep * _SEQ + j
                for l in range(_EMBED // _LANES):
                    sl = pl.ds(l * _LANES, _LANES)
                    rows_v[r, sl] = rows_v[r, sl] + pos_regs[l]
            return carry
        lax.fori_loop(0, _SEQ, jbody, 0, unroll=2)

    pltpu.sync_copy(wp_hbm.at[pl.ds(0, _SEQ)], pos_v)

    idx_copy(0, idx_a, sem_ia).start()
    idx_copy(0, idx_a, sem_ia).wait()
    for d in gather_copies(idx_a, rows_a, sem_ga):
        d.start()
    idx_copy(1, idx_b, sem_ib).start()

    def process(c, idx_x, rows_x, sem_ix, sem_gx, sem_ox,
                idx_y, rows_y, sem_iy, sem_gy, sem_oy):
        for d in gather_copies(idx_x, rows_x, sem_gx):
            d.wait()

        @pl.when(c + 2 < n_chunks)
        def _():
            idx_copy(c + 2, idx_x, sem_ix).start()

        @pl.when(c + 1 < n_chunks)
        def _():
            idx_copy(c + 1, idx_y, sem_iy).wait()

            @pl.when(c >= 1)
            def _():
                out_copy(c - 1, rows_y, sem_oy).wait()

            for d in gather_copies(idx_y, rows_y, sem_gy):
                d.start()

        add_pos(rows_x)
        out_copy(c, rows_x, sem_ox).start()

    def pair_body(t, carry):
        c0 = 2 * t
        process(c0, idx_a, rows_a, sem_ia, sem_ga, sem_oa,
                idx_b, rows_b, sem_ib, sem_gb, sem_ob)
        process(c0 + 1, idx_b, rows_b, sem_ib, sem_gb, sem_ob,
                idx_a, rows_a, sem_ia, sem_ga, sem_oa)
        return carry

    lax.fori_loop(0, n_chunks // 2, pair_body, 0)

    out_copy(n_chunks - 2, rows_a, sem_oa).wait()
    out_copy(n_chunks - 1, rows_b, sem_ob).wait()


def kernel(input, W_tok, W_pos):
    batch, seq = input.shape
    n_rows = batch * seq
    ids_flat = input.reshape(n_rows // _IDX_MINOR, _IDX_MINOR).astype(jnp.int32)

    mesh = plsc.VectorSubcoreMesh(core_axis_name="c", subcore_axis_name="s",
                                  num_cores=_NC, num_subcores=_NS)
    out = pl.kernel(
        functools.partial(_sc_body, n_rows),
        out_type=jax.ShapeDtypeStruct((n_rows, _EMBED), jnp.float32),
        mesh=mesh,
        scratch_types=[
            pltpu.VMEM((_SEQ, _EMBED), jnp.float32),
            pltpu.VMEM((_CR, _IDX_MINOR), jnp.int32),
            pltpu.VMEM((_CR, _IDX_MINOR), jnp.int32),
            pltpu.VMEM((_CHUNK_ROWS, _EMBED), jnp.float32),
            pltpu.VMEM((_CHUNK_ROWS, _EMBED), jnp.float32),
            pltpu.SemaphoreType.DMA,
            pltpu.SemaphoreType.DMA,
            pltpu.SemaphoreType.DMA,
            pltpu.SemaphoreType.DMA,
            pltpu.SemaphoreType.DMA,
            pltpu.SemaphoreType.DMA,
        ],
        compiler_params=pltpu.CompilerParams(use_tc_tiling_on_sc=False),
    )(W_tok, W_pos, ids_flat)
    return out.reshape(batch, seq, _EMBED)


# speedup vs baseline: 1.6791x; 1.0730x over previous
import functools

import jax
import jax.numpy as jnp
from jax import lax
from jax.experimental import pallas as pl
from jax.experimental.pallas import tpu as pltpu
from jax.experimental.pallas import tpu_sc as plsc

_VOCAB = 100000
_EMBED = 64
_SEQ = 200

_NC = 2
_NS = 16
_NW = _NC * _NS
_LANES = 16

_IDX_MINOR = 100
_SEQ_PER_CHUNK = 4
_CHUNK_ROWS = _SEQ_PER_CHUNK * _SEQ
_CR = _CHUNK_ROWS // _IDX_MINOR


def _sc_body(batch, wt_hbm, wp_hbm, idx_hbm, out_hbm,
             pos_v, idx_a, idx_b, rows_a, rows_b,
             sem_ia, sem_ib, sem_ga, sem_gb, sem_oa, sem_ob):
    seq_per_w = batch * _SEQ // _NW // _SEQ
    n_chunks = seq_per_w // _SEQ_PER_CHUNK
    idx_rows_per_w = seq_per_w * _SEQ // _IDX_MINOR

    wid = lax.axis_index("s") * _NC + lax.axis_index("c")

    def idx_copy(c, idx_v, sem):
        irow = wid * idx_rows_per_w + c * _CR
        return pltpu.make_async_copy(idx_hbm.at[pl.ds(irow, _CR)], idx_v, sem)

    def gather_copies(idx_v, rows_v, sem):
        return [pltpu.make_async_copy(
                    wt_hbm.at[idx_v.at[i]],
                    rows_v.at[i // 2, pl.ds((i % 2) * _IDX_MINOR, _IDX_MINOR)],
                    sem)
                for i in range(_CR)]

    def out_copy(c, rows_v, sem):
        seq_base = wid * seq_per_w + c * _SEQ_PER_CHUNK
        return pltpu.make_async_copy(
            rows_v, out_hbm.at[pl.ds(seq_base, _SEQ_PER_CHUNK)], sem)

    def add_pos(rows_v):
        def jbody(j, carry):
            pos_regs = [pos_v[j, pl.ds(l * _LANES, _LANES)]
                        for l in range(_EMBED // _LANES)]
            for b in range(_SEQ_PER_CHUNK):
                for l in range(_EMBED // _LANES):
                    sl = pl.ds(l * _LANES, _LANES)
                    rows_v[b, j, sl] = rows_v[b, j, sl] + pos_regs[l]
            return carry
        lax.fori_loop(0, _SEQ, jbody, 0, unroll=2)

    pltpu.sync_copy(wp_hbm, pos_v)

    idx_copy(0, idx_a, sem_ia).start()
    idx_copy(0, idx_a, sem_ia).wait()
    for d in gather_copies(idx_a, rows_a, sem_ga):
        d.start()
    idx_copy(1, idx_b, sem_ib).start()

    def process(c, idx_x, rows_x, sem_ix, sem_gx, sem_ox,
                idx_y, rows_y, sem_iy, sem_gy, sem_oy):
        for d in gather_copies(idx_x, rows_x, sem_gx):
            d.wait()

        @pl.when(c + 2 < n_chunks)
        def _():
            idx_copy(c + 2, idx_x, sem_ix).start()

        @pl.when(c + 1 < n_chunks)
        def _():
            idx_copy(c + 1, idx_y, sem_iy).wait()

            @pl.when(c >= 1)
            def _():
                out_copy(c - 1, rows_y, sem_oy).wait()

            for d in gather_copies(idx_y, rows_y, sem_gy):
                d.start()

        add_pos(rows_x)
        out_copy(c, rows_x, sem_ox).start()

    def pair_body(t, carry):
        c0 = 2 * t
        process(c0, idx_a, rows_a, sem_ia, sem_ga, sem_oa,
                idx_b, rows_b, sem_ib, sem_gb, sem_ob)
        process(c0 + 1, idx_b, rows_b, sem_ib, sem_gb, sem_ob,
                idx_a, rows_a, sem_ia, sem_ga, sem_oa)
        return carry

    lax.fori_loop(0, n_chunks // 2, pair_body, 0)

    out_copy(n_chunks - 2, rows_a, sem_oa).wait()
    out_copy(n_chunks - 1, rows_b, sem_ob).wait()


def kernel(input, W_tok, W_pos):
    batch, seq = input.shape
    n_rows = batch * seq
    ids_flat = input.reshape(n_rows // _IDX_MINOR, _IDX_MINOR).astype(jnp.int32)

    mesh = plsc.VectorSubcoreMesh(core_axis_name="c", subcore_axis_name="s",
                                  num_cores=_NC, num_subcores=_NS)
    out = pl.kernel(
        functools.partial(_sc_body, batch),
        out_type=jax.ShapeDtypeStruct((batch, seq, _EMBED), jnp.float32),
        mesh=mesh,
        scratch_types=[
            pltpu.VMEM((_SEQ, _EMBED), jnp.float32),
            pltpu.VMEM((_CR, _IDX_MINOR), jnp.int32),
            pltpu.VMEM((_CR, _IDX_MINOR), jnp.int32),
            pltpu.VMEM((_SEQ_PER_CHUNK, _SEQ, _EMBED), jnp.float32),
            pltpu.VMEM((_SEQ_PER_CHUNK, _SEQ, _EMBED), jnp.float32),
            pltpu.SemaphoreType.DMA,
            pltpu.SemaphoreType.DMA,
            pltpu.SemaphoreType.DMA,
            pltpu.SemaphoreType.DMA,
            pltpu.SemaphoreType.DMA,
            pltpu.SemaphoreType.DMA,
        ],
        compiler_params=pltpu.CompilerParams(use_tc_tiling_on_sc=False),
    )(W_tok, W_pos[:_SEQ], ids_flat)
    return out


# speedup vs baseline: 2.9396x; 1.7508x over previous
import functools

import jax
import jax.numpy as jnp
from jax import lax
from jax.experimental import pallas as pl
from jax.experimental.pallas import tpu as pltpu
from jax.experimental.pallas import tpu_sc as plsc

_VOCAB = 100000
_EMBED = 64
_SEQ = 200

_NC = 2
_NS = 16
_NW = _NC * _NS
_LANES = 16

_IDX_MINOR = 100
_SEQ_PER_CHUNK = 4
_CHUNK_ROWS = _SEQ_PER_CHUNK * _SEQ
_CR = _CHUNK_ROWS // _IDX_MINOR


def _sc_body(batch, wt_hbm, wp_hbm, idx_hbm, out_hbm,
             pos_v, idx_a, idx_b, rows_a, rows_b,
             sem_ia, sem_ib, sem_ga, sem_gb, sem_oa, sem_ob):
    seq_per_w = batch * _SEQ // _NW // _SEQ
    n_chunks = seq_per_w // _SEQ_PER_CHUNK
    idx_rows_per_w = seq_per_w * _SEQ // _IDX_MINOR

    wid = lax.axis_index("s") * _NC + lax.axis_index("c")

    def idx_copy(c, idx_v, sem):
        irow = wid * idx_rows_per_w + c * _CR
        return pltpu.make_async_copy(idx_hbm.at[pl.ds(irow, _CR)], idx_v, sem)

    def gather_copies(idx_v, rows_v, sem):
        return [pltpu.make_async_copy(
                    wt_hbm.at[idx_v.at[i]],
                    rows_v.at[i // 2, pl.ds((i % 2) * _IDX_MINOR, _IDX_MINOR)],
                    sem)
                for i in range(_CR)]

    def out_copy(c, rows_v, sem):
        seq_base = wid * seq_per_w + c * _SEQ_PER_CHUNK
        return pltpu.make_async_copy(
            rows_v,
            out_hbm.at[pl.ds(seq_base, _SEQ_PER_CHUNK), :, pl.ds(0, _EMBED)],
            sem)

    def add_pos(rows_v):
        def jbody(j, carry):
            pos_regs = [pos_v[j, pl.ds(l * _LANES, _LANES)]
                        for l in range(_EMBED // _LANES)]
            for b in range(_SEQ_PER_CHUNK):
                for l in range(_EMBED // _LANES):
                    sl = pl.ds(l * _LANES, _LANES)
                    rows_v[b, j, sl] = rows_v[b, j, sl] + pos_regs[l]
            return carry
        lax.fori_loop(0, _SEQ, jbody, 0, unroll=2)

    pltpu.sync_copy(wp_hbm, pos_v)

    idx_copy(0, idx_a, sem_ia).start()
    idx_copy(0, idx_a, sem_ia).wait()
    for d in gather_copies(idx_a, rows_a, sem_ga):
        d.start()
    idx_copy(1, idx_b, sem_ib).start()

    def process(c, idx_x, rows_x, sem_ix, sem_gx, sem_ox,
                idx_y, rows_y, sem_iy, sem_gy, sem_oy):
        for d in gather_copies(idx_x, rows_x, sem_gx):
            d.wait()

        @pl.when(c + 2 < n_chunks)
        def _():
            idx_copy(c + 2, idx_x, sem_ix).start()

        @pl.when(c + 1 < n_chunks)
        def _():
            idx_copy(c + 1, idx_y, sem_iy).wait()

            @pl.when(c >= 1)
            def _():
                out_copy(c - 1, rows_y, sem_oy).wait()

            for d in gather_copies(idx_y, rows_y, sem_gy):
                d.start()

        add_pos(rows_x)
        out_copy(c, rows_x, sem_ox).start()

    def pair_body(t, carry):
        c0 = 2 * t
        process(c0, idx_a, rows_a, sem_ia, sem_ga, sem_oa,
                idx_b, rows_b, sem_ib, sem_gb, sem_ob)
        process(c0 + 1, idx_b, rows_b, sem_ib, sem_gb, sem_ob,
                idx_a, rows_a, sem_ia, sem_ga, sem_oa)
        return carry

    lax.fori_loop(0, n_chunks // 2, pair_body, 0)

    out_copy(n_chunks - 2, rows_a, sem_oa).wait()
    out_copy(n_chunks - 1, rows_b, sem_ob).wait()


def kernel(input, W_tok, W_pos):
    batch, seq = input.shape
    n_rows = batch * seq
    ids_flat = input.reshape(n_rows // _IDX_MINOR, _IDX_MINOR).astype(jnp.int32)

    mesh = plsc.VectorSubcoreMesh(core_axis_name="c", subcore_axis_name="s",
                                  num_cores=_NC, num_subcores=_NS)
    out = pl.kernel(
        functools.partial(_sc_body, batch),
        out_type=jax.ShapeDtypeStruct((batch, seq, 2 * _EMBED), jnp.float32),
        mesh=mesh,
        scratch_types=[
            pltpu.VMEM((_SEQ, _EMBED), jnp.float32),
            pltpu.VMEM((_CR, _IDX_MINOR), jnp.int32),
            pltpu.VMEM((_CR, _IDX_MINOR), jnp.int32),
            pltpu.VMEM((_SEQ_PER_CHUNK, _SEQ, _EMBED), jnp.float32),
            pltpu.VMEM((_SEQ_PER_CHUNK, _SEQ, _EMBED), jnp.float32),
            pltpu.SemaphoreType.DMA,
            pltpu.SemaphoreType.DMA,
            pltpu.SemaphoreType.DMA,
            pltpu.SemaphoreType.DMA,
            pltpu.SemaphoreType.DMA,
            pltpu.SemaphoreType.DMA,
        ],
        compiler_params=pltpu.CompilerParams(use_tc_tiling_on_sc=False),
    )(W_tok, W_pos[:_SEQ], ids_flat)
    return out[:, :, :_EMBED]
